# baseline (speedup 1.0000x reference)
import math

import jax
import jax.numpy as jnp
from jax.experimental import pallas as pl

K = 1000
S = 8
D = 1024
B = 4096
BETA = 5.0
ALPHA = 2.0
_C = math.log(ALPHA) - BETA

_BB = 512


def _tda_kernel(x_ref, m_ref, o_ref):
    s = pl.program_id(1)
    a = jax.lax.dot_general(
        x_ref[...], m_ref[0],
        dimension_numbers=(((1,), (1,)), ((), ())),
        preferred_element_type=jnp.float32,
    )
    e = jnp.exp(BETA * a + _C)

    @pl.when(s == 0)
    def _init():
        o_ref[...] = e

    @pl.when(s != 0)
    def _acc():
        o_ref[...] += e


def kernel(x, memory):
    mem_r = jnp.transpose(memory, (1, 0, 2)).astype(jnp.bfloat16)
    x16 = x.astype(jnp.bfloat16)
    grid = (B // _BB, S)
    return pl.pallas_call(
        _tda_kernel,
        grid=grid,
        in_specs=[
            pl.BlockSpec((_BB, D), lambda i, s: (i, 0)),
            pl.BlockSpec((1, K, D), lambda i, s: (s, 0, 0)),
        ],
        out_specs=pl.BlockSpec((_BB, K), lambda i, s: (i, 0)),
        out_shape=jax.ShapeDtypeStruct((B, K), jnp.float32),
    )(x16, mem_r)


# speedup vs baseline: 1.1627x; 1.1627x over previous
import math

import jax
import jax.numpy as jnp
from jax.experimental import pallas as pl

K = 1000
S = 8
D = 1024
B = 4096
BETA = 5.0
ALPHA = 2.0
_XSCALE = BETA * math.log2(math.e)
_OSCALE = ALPHA * math.exp(-BETA)

_BB = 2048


def _tda_kernel(x_ref, m_ref, o_ref):
    s = pl.program_id(1)
    a = jax.lax.dot_general(
        x_ref[...], m_ref[0],
        dimension_numbers=(((1,), (1,)), ((), ())),
        preferred_element_type=jnp.float32,
    )
    e = jnp.exp2(a)

    @pl.when(s == 0)
    def _init():
        o_ref[...] = e

    @pl.when((s != 0) & (s != S - 1))
    def _acc():
        o_ref[...] += e

    @pl.when(s == S - 1)
    def _fin():
        o_ref[...] = (o_ref[...] + e) * _OSCALE


def kernel(x, memory):
    mem_r = jnp.transpose(memory, (1, 0, 2)).astype(jnp.bfloat16)
    x16 = (x * _XSCALE).astype(jnp.bfloat16)
    grid = (B // _BB, S)
    return pl.pallas_call(
        _tda_kernel,
        grid=grid,
        in_specs=[
            pl.BlockSpec((_BB, D), lambda i, s: (i, 0)),
            pl.BlockSpec((1, K, D), lambda i, s: (s, 0, 0)),
        ],
        out_specs=pl.BlockSpec((_BB, K), lambda i, s: (i, 0)),
        out_shape=jax.ShapeDtypeStruct((B, K), jnp.float32),
    )(x16, mem_r)


# speedup vs baseline: 1.2759x; 1.0974x over previous
import math

import jax
import jax.numpy as jnp
from jax.experimental import pallas as pl

K = 1000
S = 8
D = 1024
B = 4096
BETA = 5.0
ALPHA = 2.0
_XSCALE = BETA * math.log2(math.e)
_OSCALE = ALPHA * math.exp(-BETA)

_BB = 2048


def _tda_kernel(x_ref, m_ref, o_ref):
    xb = x_ref[...]
    acc = None
    for s in range(S):
        a = jax.lax.dot_general(
            xb, m_ref[s],
            dimension_numbers=(((1,), (1,)), ((), ())),
            preferred_element_type=jnp.float32,
        )
        e = jnp.exp2(a)
        acc = e if acc is None else acc + e
    o_ref[...] = acc * _OSCALE


def kernel(x, memory):
    mem_r = jnp.transpose(memory, (1, 0, 2)).astype(jnp.bfloat16)
    x16 = (x * _XSCALE).astype(jnp.bfloat16)
    grid = (B // _BB,)
    return pl.pallas_call(
        _tda_kernel,
        grid=grid,
        in_specs=[
            pl.BlockSpec((_BB, D), lambda i: (i, 0)),
            pl.BlockSpec((S, K, D), lambda i: (0, 0, 0)),
        ],
        out_specs=pl.BlockSpec((_BB, K), lambda i: (i, 0)),
        out_shape=jax.ShapeDtypeStruct((B, K), jnp.float32),
    )(x16, mem_r)
